# Initial kernel scaffold; baseline (speedup 1.0000x reference)
import jax
import jax.numpy as jnp
from jax.experimental import pallas as pl


def kernel(x, enc_w, enc_b, dec_w, dec_b, codebooks):
    raise NotImplementedError("write your pallas kernel here")


# speedup vs baseline: 1.6045x; 1.6045x over previous
import jax
import jax.numpy as jnp
from jax.experimental import pallas as pl
from jax.experimental.pallas import tpu as pltpu

NQ = 8
K = 1024
D = 256
W = 2048


def _encode_kernel(xtaps_ref, enc_w_ref, enc_b_ref, o_ref):
    o_ref[...] = jax.lax.dot(
        xtaps_ref[...].astype(jnp.bfloat16),
        enc_w_ref[...].astype(jnp.bfloat16),
        preferred_element_type=jnp.float32) + enc_b_ref[...]


def _stage_kernel(res_ref, rnorm_ref, cnorm_ref, cbt_ref, cb_ref,
                  res_out_ref, loss_ref):
    residual = res_ref[...]
    cb = cb_ref[...]
    cb_hi = cb.astype(jnp.bfloat16)
    r1 = cb - cb_hi.astype(jnp.float32)
    cb_mid = r1.astype(jnp.bfloat16)
    cb_lo = (r1 - cb_mid.astype(jnp.float32)).astype(jnp.bfloat16)
    scores = jax.lax.dot(residual.astype(jnp.bfloat16),
                         cbt_ref[...].astype(jnp.bfloat16),
                         preferred_element_type=jnp.float32)
    d = rnorm_ref[...] - 2.0 * scores + cnorm_ref[...]
    m = jnp.min(d, axis=1, keepdims=True)
    iota = jax.lax.broadcasted_iota(jnp.int32, (W, K), 1)
    idx = jnp.min(jnp.where(d == m, iota, K), axis=1, keepdims=True)
    onehot = (iota == idx).astype(jnp.bfloat16)
    qv = (jax.lax.dot(onehot, cb_hi,
                      preferred_element_type=jnp.float32)
          + jax.lax.dot(onehot, cb_mid,
                        preferred_element_type=jnp.float32)
          + jax.lax.dot(onehot, cb_lo,
                        preferred_element_type=jnp.float32))
    new_res = residual - qv
    res_out_ref[...] = new_res
    loss_ref[0] = jnp.sum(new_res * new_res, keepdims=True)


def _decode_kernel(q_ref, dec_w_ref, y_ref):
    y_ref[...] = jax.lax.dot(q_ref[...], dec_w_ref[...],
                             precision=jax.lax.Precision.HIGHEST,
                             preferred_element_type=jnp.float32)


@jax.jit
def kernel(x, enc_w, enc_b, dec_w, dec_b, codebooks):
    B, _, T = x.shape
    N = B * T
    nb = N // W
    xt = x[:, 0, :]
    left = jnp.pad(xt[:, :-1], ((0, 0), (1, 0)))
    right = jnp.pad(xt[:, 1:], ((0, 0), (0, 1)))
    xtaps = jnp.stack([left, xt, right], axis=-1).reshape(N, 3)
    enc_wr = enc_w[:, 0, :].T
    enc_b2 = enc_b[None, :]
    dec_wr = dec_w[0]
    cbt = jnp.transpose(codebooks, (0, 2, 1))

    encoded = pl.pallas_call(
        _encode_kernel,
        grid=(nb,),
        in_specs=[
            pl.BlockSpec((W, 3), lambda i: (i, 0)),
            pl.BlockSpec((3, D), lambda i: (0, 0)),
            pl.BlockSpec((1, D), lambda i: (0, 0)),
        ],
        out_specs=pl.BlockSpec((W, D), lambda i: (i, 0)),
        out_shape=jax.ShapeDtypeStruct((N, D), jnp.float32),
    )(xtaps, enc_wr, enc_b2)

    residual = encoded
    loss_total = jnp.zeros((), jnp.float32)
    for q in range(NQ):
        rnorm = jnp.sum(residual.reshape(B, T, D) ** 2,
                        axis=-1, keepdims=True).reshape(N, 1)
        cnorm = jnp.sum(codebooks[q] ** 2, axis=-1)[None, :]
        residual, loss_parts = pl.pallas_call(
            _stage_kernel,
            grid=(nb,),
            in_specs=[
                pl.BlockSpec((W, D), lambda i: (i, 0)),
                pl.BlockSpec((W, 1), lambda i: (i, 0)),
                pl.BlockSpec((1, K), lambda i: (0, 0)),
                pl.BlockSpec((D, K), lambda i: (0, 0)),
                pl.BlockSpec((K, D), lambda i: (0, 0)),
            ],
            out_specs=[
                pl.BlockSpec((W, D), lambda i: (i, 0)),
                pl.BlockSpec((1, 1, 1), lambda i: (i, 0, 0)),
            ],
            out_shape=[
                jax.ShapeDtypeStruct((N, D), jnp.float32),
                jax.ShapeDtypeStruct((nb, 1, 1), jnp.float32),
            ],
        )(residual, rnorm, cnorm, cbt[q], codebooks[q])
        loss_total = loss_total + jnp.sum(loss_parts)

    quantized = encoded - residual
    y = pl.pallas_call(
        _decode_kernel,
        grid=(nb,),
        in_specs=[
            pl.BlockSpec((W, D), lambda i: (i, 0)),
            pl.BlockSpec((D, 3), lambda i: (0, 0)),
        ],
        out_specs=pl.BlockSpec((W, 3), lambda i: (i, 0)),
        out_shape=jax.ShapeDtypeStruct((N, 3), jnp.float32),
    )(quantized, dec_wr)

    yb = y.reshape(B, T, 3)
    decoded = (yb[:, :, 1]
               + jnp.pad(yb[:, :-1, 0], ((0, 0), (1, 0)))
               + jnp.pad(yb[:, 1:, 2], ((0, 0), (0, 1))))
    decoded = decoded[:, None, :] + dec_b[None, :, None]
    commit_loss = loss_total / jnp.float32(NQ * N * D)
    return decoded, commit_loss
